# baseline (speedup 1.0000x reference)
import jax
import jax.numpy as jnp
from jax import lax
from jax.experimental import pallas as pl
from jax.experimental.pallas import tpu as pltpu
from jax.experimental.pallas import tpu_sc as plsc

_MIN_NORM = 1e-15
_MAXNORM = 1.0 - 4e-3

_N, _E, _D = 10000, 160000, 256
_HALF = _D // 2
_LANES = 128
_CHUNK = _LANES
_NSUB = 16
_NHALF = 2
_CPH = 40
_CPS = _NHALF * _CPH
_CPR = (_N // _NSUB) // 8 * 8
_TAIL = _N - _CPR * _NSUB
_BN = 1000


def _logmap_body(x_ref, o_ref):
    x = x_ref[...]
    nrm = jnp.sqrt(jnp.sum(x * x, axis=1, keepdims=True))
    nrm = jnp.maximum(nrm, _MIN_NORM)
    t = jnp.clip(nrm, -1.0 + 1e-7, 1.0 - 1e-7)
    art = 0.5 * (jnp.log1p(t) - jnp.log1p(-t))
    xt = x * (art / nrm)
    o_ref[0] = xt[:, :_HALF]
    o_ref[1] = xt[:, _HALF:]


def _expmap_body(s_ref, o_ref):
    lo = s_ref[0]
    hi = s_ref[1]
    nrm = jnp.sqrt(jnp.sum(lo * lo, axis=1, keepdims=True)
                   + jnp.sum(hi * hi, axis=1, keepdims=True))
    nrm = jnp.maximum(nrm, _MIN_NORM)
    g = jnp.tanh(nrm) / nrm
    ylo = lo * g
    yhi = hi * g
    ynrm = jnp.sqrt(jnp.sum(ylo * ylo, axis=1, keepdims=True)
                    + jnp.sum(yhi * yhi, axis=1, keepdims=True))
    ynrm = jnp.maximum(ynrm, _MIN_NORM)
    scale = jnp.where(ynrm > _MAXNORM, _MAXNORM / ynrm, 1.0)
    o_ref[:, :_HALF] = ylo * scale
    o_ref[:, _HALF:] = yhi * scale


_PADROWS = 8
_NCHUNKS = (_E // _LANES + 7) // 8 * 8
_EPAD = _NCHUNKS * _LANES - _E
_SHORT = _NCHUNKS - (_NSUB - 1) * _CPS - _CPH


def _sc_body(table, adj_hbm, out_hbm,
             idx_row, idx_col, ra, rb, acc,
             gsem_a, gsem_b, ssem_a, ssem_b):
    c = lax.axis_index("c")
    s = lax.axis_index("s")
    tbl = table.at[c]

    def gather(t, r, sem):
        pltpu.async_copy(tbl.at[idx_col.at[t]], r, sem)

    def wait_gather(t, r, sem):
        pltpu.make_async_copy(tbl.at[idx_col.at[t]], r, sem).wait()

    def scatter(t, r, sem):
        pltpu.async_copy(r, acc.at[idx_row.at[t]], sem, add=True)

    def wait_scatter(t, r, sem):
        pltpu.make_async_copy(r, acc.at[idx_row.at[t]], sem).wait()

    def fetch_idx(h, cnt):
        off = pl.multiple_of(s * _CPS + h * _CPH, 8)
        pltpu.sync_copy(adj_hbm.at[0, pl.ds(off, cnt)],
                        idx_row.at[pl.ds(0, cnt)])
        pltpu.sync_copy(adj_hbm.at[1, pl.ds(off, cnt)],
                        idx_col.at[pl.ds(0, cnt)])

    fetch_idx(0, _CPH)
    gather(0, ra, gsem_a)

    def zfill(j, cc):
        for jj in range(16):
            for k in range(_HALF // 16):
                rb[j * 16 + jj, pl.ds(k * 16, 16)] = jnp.zeros(
                    (16,), jnp.float32)
        return cc

    lax.fori_loop(0, _LANES // 16, zfill, 0)
    base = s * _CPR
    for k in range(_CPR // _LANES):
        pltpu.sync_copy(rb, acc.at[pl.ds(base + k * _LANES, _LANES)])
    rem = _CPR % _LANES
    pltpu.sync_copy(rb.at[pl.ds(0, rem)],
                    acc.at[pl.ds(base + _CPR - rem, rem)])

    @pl.when(s == _NSUB - 1)
    def _zero_tail():
        pltpu.sync_copy(rb.at[pl.ds(0, _TAIL + _PADROWS)],
                        acc.at[pl.ds(_CPR * _NSUB, _TAIL + _PADROWS)])

    gather(1, rb, gsem_b)
    plsc.subcore_barrier()

    def half_pass(h, double_primed, ncnk):
        if not double_primed:
            fetch_idx(h, ncnk)
            gather(0, ra, gsem_a)

        def pair(p, cc):
            ta = 2 * p
            tb = 2 * p + 1

            @pl.when(p > 0)
            def _():
                wait_scatter(tb, rb, ssem_b)
            if double_primed:
                @pl.when(p > 0)
                def _():
                    gather(tb, rb, gsem_b)
            else:
                gather(tb, rb, gsem_b)
            wait_gather(ta, ra, gsem_a)
            scatter(ta, ra, ssem_a)

            wait_scatter(ta, ra, ssem_a)

            @pl.when(p < ncnk // 2 - 1)
            def _():
                gather(ta + 2, ra, gsem_a)
            wait_gather(tb, rb, gsem_b)
            scatter(tb, rb, ssem_b)
            return cc

        lax.fori_loop(0, ncnk // 2, pair, 0)
        wait_scatter(ncnk - 1, rb, ssem_b)

    half_pass(0, True, _CPH)

    @pl.when(s < _NSUB - 1)
    def _full_second():
        half_pass(1, False, _CPH)

    @pl.when(s == _NSUB - 1)
    def _short_second():
        half_pass(1, False, _SHORT)

    plsc.subcore_barrier()

    pltpu.sync_copy(acc.at[pl.ds(base, _CPR)],
                    out_hbm.at[c, pl.ds(base, _CPR)])

    @pl.when(s == _NSUB - 1)
    def _out_tail():
        pltpu.sync_copy(acc.at[pl.ds(_CPR * _NSUB, _TAIL)],
                        out_hbm.at[c, pl.ds(_CPR * _NSUB, _TAIL)])


def kernel(x, adj):
    n, d = x.shape
    xt2 = pl.pallas_call(
        _logmap_body,
        grid=(n // _BN,),
        in_specs=[pl.BlockSpec((_BN, d), lambda i: (i, 0))],
        out_specs=pl.BlockSpec((2, _BN, _HALF), lambda i: (0, i, 0)),
        out_shape=jax.ShapeDtypeStruct((2, n, _HALF), jnp.float32),
    )(x)

    ar = jnp.arange(_EPAD, dtype=jnp.int32)
    adj_p = jnp.concatenate(
        [adj, jnp.stack([n + lax.rem(ar, _PADROWS), ar])], axis=1)

    mesh = plsc.VectorSubcoreMesh(core_axis_name="c", subcore_axis_name="s")
    support2 = pl.kernel(
        _sc_body,
        out_type=jax.ShapeDtypeStruct((2, n, _HALF), jnp.float32),
        mesh=mesh,
        scratch_types=[
            pltpu.VMEM((_CPH, _LANES), jnp.int32),
            pltpu.VMEM((_CPH, _LANES), jnp.int32),
            pltpu.VMEM((_LANES, _HALF), jnp.float32),
            pltpu.VMEM((_LANES, _HALF), jnp.float32),
            pltpu.VMEM_SHARED((n + _PADROWS, _HALF), jnp.float32),
            pltpu.SemaphoreType.DMA,
            pltpu.SemaphoreType.DMA,
            pltpu.SemaphoreType.DMA,
            pltpu.SemaphoreType.DMA,
        ],
    )(xt2, adj_p.reshape(2, _NCHUNKS, _LANES))

    out = pl.pallas_call(
        _expmap_body,
        grid=(n // _BN,),
        in_specs=[pl.BlockSpec((2, _BN, _HALF), lambda i: (0, i, 0))],
        out_specs=pl.BlockSpec((_BN, d), lambda i: (i, 0)),
        out_shape=jax.ShapeDtypeStruct((n, d), jnp.float32),
    )(support2)
    return out


# speedup vs baseline: 1.0290x; 1.0290x over previous
import jax
import jax.numpy as jnp
from jax import lax
from jax.experimental import pallas as pl
from jax.experimental.pallas import tpu as pltpu
from jax.experimental.pallas import tpu_sc as plsc

_MIN_NORM = 1e-15
_MAXNORM = 1.0 - 4e-3

_N, _E, _D = 10000, 160000, 256
_HALF = _D // 2
_LANES = 128
_CHUNK = _LANES
_NSUB = 16
_NHALF = 2
_CPH = 40
_CPS = _NHALF * _CPH
_CPR = (_N // _NSUB) // 8 * 8
_TAIL = _N - _CPR * _NSUB
_BN = 2000


def _logmap_body(x_ref, o_ref):
    x = x_ref[...]
    nrm = jnp.sqrt(jnp.sum(x * x, axis=1, keepdims=True))
    nrm = jnp.maximum(nrm, _MIN_NORM)
    t = jnp.clip(nrm, -1.0 + 1e-7, 1.0 - 1e-7)
    art = 0.5 * (jnp.log1p(t) - jnp.log1p(-t))
    xt = x * (art / nrm)
    o_ref[0] = xt[:, :_HALF]
    o_ref[1] = xt[:, _HALF:]


def _expmap_body(s_ref, o_ref):
    lo = s_ref[0]
    hi = s_ref[1]
    nrm = jnp.sqrt(jnp.sum(lo * lo, axis=1, keepdims=True)
                   + jnp.sum(hi * hi, axis=1, keepdims=True))
    nrm = jnp.maximum(nrm, _MIN_NORM)
    g = jnp.tanh(nrm) / nrm
    ylo = lo * g
    yhi = hi * g
    ynrm = jnp.sqrt(jnp.sum(ylo * ylo, axis=1, keepdims=True)
                    + jnp.sum(yhi * yhi, axis=1, keepdims=True))
    ynrm = jnp.maximum(ynrm, _MIN_NORM)
    scale = jnp.where(ynrm > _MAXNORM, _MAXNORM / ynrm, 1.0)
    o_ref[:, :_HALF] = ylo * scale
    o_ref[:, _HALF:] = yhi * scale


_PADROWS = 8
_NCHUNKS = (_E // _LANES + 7) // 8 * 8
_EPAD = _NCHUNKS * _LANES - _E
_SHORT = _NCHUNKS - (_NSUB - 1) * _CPS - _CPH


def _sc_body(table, adj_hbm, out_hbm,
             idx_row, idx_col, ra, rb, acc,
             gsem_a, gsem_b, ssem_a, ssem_b):
    c = lax.axis_index("c")
    s = lax.axis_index("s")
    tbl = table.at[c]

    def gather(t, r, sem):
        pltpu.async_copy(tbl.at[idx_col.at[t]], r, sem)

    def wait_gather(t, r, sem):
        pltpu.make_async_copy(tbl.at[idx_col.at[t]], r, sem).wait()

    def scatter(t, r, sem):
        pltpu.async_copy(r, acc.at[idx_row.at[t]], sem, add=True)

    def wait_scatter(t, r, sem):
        pltpu.make_async_copy(r, acc.at[idx_row.at[t]], sem).wait()

    def fetch_idx(h, cnt):
        off = pl.multiple_of(s * _CPS + h * _CPH, 8)
        pltpu.sync_copy(adj_hbm.at[0, pl.ds(off, cnt)],
                        idx_row.at[pl.ds(0, cnt)])
        pltpu.sync_copy(adj_hbm.at[1, pl.ds(off, cnt)],
                        idx_col.at[pl.ds(0, cnt)])

    fetch_idx(0, _CPH)
    gather(0, ra, gsem_a)

    def zfill(j, cc):
        for jj in range(16):
            for k in range(_HALF // 16):
                rb[j * 16 + jj, pl.ds(k * 16, 16)] = jnp.zeros(
                    (16,), jnp.float32)
        return cc

    lax.fori_loop(0, _LANES // 16, zfill, 0)
    base = s * _CPR
    for k in range(_CPR // _LANES):
        pltpu.sync_copy(rb, acc.at[pl.ds(base + k * _LANES, _LANES)])
    rem = _CPR % _LANES
    pltpu.sync_copy(rb.at[pl.ds(0, rem)],
                    acc.at[pl.ds(base + _CPR - rem, rem)])

    @pl.when(s == _NSUB - 1)
    def _zero_tail():
        pltpu.sync_copy(rb.at[pl.ds(0, _TAIL + _PADROWS)],
                        acc.at[pl.ds(_CPR * _NSUB, _TAIL + _PADROWS)])

    gather(1, rb, gsem_b)
    plsc.subcore_barrier()

    def half_pass(h, double_primed, ncnk):
        if not double_primed:
            fetch_idx(h, ncnk)
            gather(0, ra, gsem_a)

        def pair(p, cc):
            ta = 2 * p
            tb = 2 * p + 1

            @pl.when(p > 0)
            def _():
                wait_scatter(tb, rb, ssem_b)
            if double_primed:
                @pl.when(p > 0)
                def _():
                    gather(tb, rb, gsem_b)
            else:
                gather(tb, rb, gsem_b)
            wait_gather(ta, ra, gsem_a)
            scatter(ta, ra, ssem_a)

            wait_scatter(ta, ra, ssem_a)

            @pl.when(p < ncnk // 2 - 1)
            def _():
                gather(ta + 2, ra, gsem_a)
            wait_gather(tb, rb, gsem_b)
            scatter(tb, rb, ssem_b)
            return cc

        lax.fori_loop(0, ncnk // 2, pair, 0)
        wait_scatter(ncnk - 1, rb, ssem_b)

    half_pass(0, True, _CPH)

    @pl.when(s < _NSUB - 1)
    def _full_second():
        half_pass(1, False, _CPH)

    @pl.when(s == _NSUB - 1)
    def _short_second():
        half_pass(1, False, _SHORT)

    plsc.subcore_barrier()

    pltpu.sync_copy(acc.at[pl.ds(base, _CPR)],
                    out_hbm.at[c, pl.ds(base, _CPR)])

    @pl.when(s == _NSUB - 1)
    def _out_tail():
        pltpu.sync_copy(acc.at[pl.ds(_CPR * _NSUB, _TAIL)],
                        out_hbm.at[c, pl.ds(_CPR * _NSUB, _TAIL)])


def kernel(x, adj):
    n, d = x.shape
    xt2 = pl.pallas_call(
        _logmap_body,
        grid=(n // _BN,),
        in_specs=[pl.BlockSpec((_BN, d), lambda i: (i, 0))],
        out_specs=pl.BlockSpec((2, _BN, _HALF), lambda i: (0, i, 0)),
        out_shape=jax.ShapeDtypeStruct((2, n, _HALF), jnp.float32),
    )(x)

    ar = jnp.arange(_EPAD, dtype=jnp.int32)
    adj_p = jnp.concatenate(
        [adj, jnp.stack([n + lax.rem(ar, _PADROWS), ar])], axis=1)

    mesh = plsc.VectorSubcoreMesh(core_axis_name="c", subcore_axis_name="s")
    support2 = pl.kernel(
        _sc_body,
        out_type=jax.ShapeDtypeStruct((2, n, _HALF), jnp.float32),
        mesh=mesh,
        scratch_types=[
            pltpu.VMEM((_CPH, _LANES), jnp.int32),
            pltpu.VMEM((_CPH, _LANES), jnp.int32),
            pltpu.VMEM((_LANES, _HALF), jnp.float32),
            pltpu.VMEM((_LANES, _HALF), jnp.float32),
            pltpu.VMEM_SHARED((n + _PADROWS, _HALF), jnp.float32),
            pltpu.SemaphoreType.DMA,
            pltpu.SemaphoreType.DMA,
            pltpu.SemaphoreType.DMA,
            pltpu.SemaphoreType.DMA,
        ],
    )(xt2, adj_p.reshape(2, _NCHUNKS, _LANES))

    out = pl.pallas_call(
        _expmap_body,
        grid=(n // _BN,),
        in_specs=[pl.BlockSpec((2, _BN, _HALF), lambda i: (0, i, 0))],
        out_specs=pl.BlockSpec((_BN, d), lambda i: (i, 0)),
        out_shape=jax.ShapeDtypeStruct((n, d), jnp.float32),
    )(support2)
    return out
